# Initial kernel scaffold; baseline (speedup 1.0000x reference)
import jax
import jax.numpy as jnp
from jax.experimental import pallas as pl


def kernel(input):
    raise NotImplementedError("write your pallas kernel here")


# speedup vs baseline: 6.8061x; 6.8061x over previous
import jax
import jax.numpy as jnp
from jax import lax
from jax.experimental import pallas as pl
from jax.experimental.pallas import tpu as pltpu
from jax.experimental.pallas import tpu_sc as plsc

_H = 224
_W = 224
_K = 16
_LH = 14
_LW = 14
_KK = _K * _K
_L = _LH * _LW
_SLAB = _KK * _L


def _fold_body(in_hbm, out_hbm, in_buf, out_buf):
    info = plsc.get_sparse_core_info()
    nc, ns = info.num_cores, info.num_subcores
    nw = nc * ns
    wid = lax.axis_index("s") * nc + lax.axis_index("c")
    per_w = in_hbm.shape[0] // nw
    lanes = lax.iota(jnp.int32, 16) * _L

    def do_slab(i, carry):
        slab = wid * per_w + i
        pltpu.sync_copy(in_hbm.at[slab], in_buf)

        def row(t, c):
            oh = t // _K
            kh = t - oh * _K
            src0 = kh * (_K * _L) + oh * _LW
            dst0 = t * _W
            for ow in range(_LW):
                idx = lanes + (src0 + ow)
                out_buf[pl.ds(dst0 + ow * _K, _K)] = plsc.load_gather(
                    in_buf, [idx]
                )
            return c

        lax.fori_loop(0, _LH * _K, row, 0)
        pltpu.sync_copy(out_buf, out_hbm.at[slab])
        return carry

    lax.fori_loop(0, per_w, do_slab, 0)


def kernel(input):
    B, CK, _ = input.shape
    C = CK // _KK
    flat_in = input.reshape(B * C, _SLAB)
    mesh = plsc.VectorSubcoreMesh(core_axis_name="c", subcore_axis_name="s")
    out = pl.kernel(
        _fold_body,
        out_type=jax.ShapeDtypeStruct((B * C, _SLAB), jnp.float32),
        mesh=mesh,
        scratch_types=[
            pltpu.VMEM((_SLAB,), jnp.float32),
            pltpu.VMEM((_SLAB,), jnp.float32),
        ],
        compiler_params=pltpu.CompilerParams(needs_layout_passes=False),
    )(flat_in)
    return out.reshape(B, C, _H, _W)


# speedup vs baseline: 8.7463x; 1.2851x over previous
import jax
import jax.numpy as jnp
from jax import lax
from jax.experimental import pallas as pl
from jax.experimental.pallas import tpu as pltpu
from jax.experimental.pallas import tpu_sc as plsc

_H = 224
_W = 224
_K = 16
_LH = 14
_LW = 14
_KK = _K * _K
_L = _LH * _LW
_SLAB = _KK * _L


def _fold_body(in_hbm, out_hbm, in_buf, out_buf):
    info = plsc.get_sparse_core_info()
    nc, ns = info.num_cores, info.num_subcores
    nw = nc * ns
    wid = lax.axis_index("s") * nc + lax.axis_index("c")
    per_w = in_hbm.shape[0] // nw
    lanes = lax.iota(jnp.int32, 16) * _L
    vecs = [lanes + ow for ow in range(_LW)]

    def do_slab(i, carry):
        slab = wid * per_w + i
        pltpu.sync_copy(in_hbm.at[slab], in_buf)

        @plsc.parallel_loop(0, _LH)
        def row(oh):
            s_oh = oh * _LW
            d_oh = oh * (_K * _W)
            for kh in range(_K):
                s = s_oh + kh * (_K * _L)
                d = d_oh + kh * _W
                for ow in range(_LW):
                    out_buf[pl.ds(d + ow * _K, _K)] = plsc.load_gather(
                        in_buf, [vecs[ow] + s]
                    )

        pltpu.sync_copy(out_buf, out_hbm.at[slab])
        return carry

    lax.fori_loop(0, per_w, do_slab, 0)


def kernel(input):
    B, CK, _ = input.shape
    C = CK // _KK
    flat_in = input.reshape(B * C, _SLAB)
    mesh = plsc.VectorSubcoreMesh(core_axis_name="c", subcore_axis_name="s")
    out = pl.kernel(
        _fold_body,
        out_type=jax.ShapeDtypeStruct((B * C, _SLAB), jnp.float32),
        mesh=mesh,
        scratch_types=[
            pltpu.VMEM((_SLAB,), jnp.float32),
            pltpu.VMEM((_SLAB,), jnp.float32),
        ],
        compiler_params=pltpu.CompilerParams(needs_layout_passes=False),
    )(flat_in)
    return out.reshape(B, C, _H, _W)
